# baseline (speedup 1.0000x reference)
import functools

import jax
import jax.numpy as jnp
from jax import lax
from jax.experimental import pallas as pl
from jax.experimental.pallas import tpu as pltpu
from jax.experimental.pallas import tpu_sc as plsc

_V = 1000000
_D = 64
_W = 512
_NBLK = 63
_TAIL = _V - 31 * _NBLK * _W


def _make_pack():
    info = plsc.get_sparse_core_info()
    mesh = plsc.VectorSubcoreMesh(core_axis_name="c", subcore_axis_name="s")

    @functools.partial(
        pl.kernel,
        mesh=mesh,
        out_type=jax.ShapeDtypeStruct((_V // 2, 128), jnp.float32),
        scratch_types=[
            pltpu.VMEM((_D, _W), jnp.float32),
            pltpu.VMEM((_D, _W), jnp.float32),
            pltpu.VMEM((_W // 2, 128), jnp.float32),
            pltpu.VMEM((_D, _TAIL), jnp.float32),
            pltpu.VMEM((_TAIL // 2, 128), jnp.float32),
            pltpu.SemaphoreType.DMA,
            pltpu.SemaphoreType.DMA,
            pltpu.SemaphoreType.DMA,
        ],
        compiler_params=pltpu.CompilerParams(use_tc_tiling_on_sc=True,
                                             needs_layout_passes=False),
    )
    def pack_kernel(t_t, p1, in0, in1, outb, tin, tout, si0, si1, so):
        w = lax.axis_index("s") * info.num_cores + lax.axis_index("c")
        iota = lax.iota(jnp.int32, 16)
        rowvecs = [iota + 16 * c for c in range(4)]

        def transform(src, dst, np_rows):
            def body(p, carry):
                for h in (0, 1):
                    col = jnp.full((16,), 2 * p + h, jnp.int32)
                    for c in range(4):
                        vals = plsc.load_gather(src, [rowvecs[c], col])
                        dst[p, pl.ds(64 * h + 16 * c, 16)] = vals
                return carry

            lax.fori_loop(0, np_rows, body, 0)

        @pl.when(w < 31)
        def _main():
            base = w * _NBLK
            ins = (in0, in1)
            sis = (si0, si1)

            def start_in(t, b):
                pltpu.async_copy(
                    t_t.at[:, pl.ds((base + t) * _W, _W)], ins[b], sis[b])

            def wait_in(b):
                pltpu.make_async_copy(
                    t_t.at[:, pl.ds(0, _W)], ins[b], sis[b]).wait()

            def wait_out():
                pltpu.make_async_copy(
                    outb, p1.at[pl.ds(0, _W // 2)], so).wait()

            start_in(0, 0)
            start_in(1, 1)

            def pair(k, carry):
                for bpar in (0, 1):
                    t = 2 * k + bpar
                    wait_in(bpar)

                    @pl.when(t > 0)
                    def _():
                        wait_out()

                    transform(ins[bpar], outb, _W // 2)

                    @pl.when(t + 2 < _NBLK)
                    def _():
                        start_in(t + 2, bpar)

                    pltpu.async_copy(
                        outb, p1.at[pl.ds((base + t) * (_W // 2), _W // 2)],
                        so)
                return carry

            lax.fori_loop(0, (_NBLK - 1) // 2, pair, 0)
            t = _NBLK - 1
            wait_in(t % 2)
            wait_out()
            transform(ins[t % 2], outb, _W // 2)
            pltpu.async_copy(
                outb, p1.at[pl.ds((base + t) * (_W // 2), _W // 2)], so)
            wait_out()

        @pl.when(w == 31)
        def _tail():
            pltpu.sync_copy(t_t.at[:, pl.ds(31 * _NBLK * _W, _TAIL)], tin)
            transform(tin, tout, _TAIL // 2)
            pltpu.sync_copy(tout, p1.at[pl.ds(31 * _NBLK * _W // 2,
                                              _TAIL // 2)])

    return pack_kernel


_BS = 16384
_S = 50
_BPW = 512


def _make_gather():
    info = plsc.get_sparse_core_info()
    mesh = plsc.VectorSubcoreMesh(core_axis_name="c", subcore_axis_name="s")

    @functools.partial(
        pl.kernel,
        mesh=mesh,
        out_type=jax.ShapeDtypeStruct((_S, 8, 128, 8, 128), jnp.float32),
        scratch_types=[
            pltpu.VMEM((_S, _BPW), jnp.int32),
            pltpu.VMEM((_BPW // 2, _D), jnp.float32),
            pltpu.VMEM((_BPW // 2, _D), jnp.float32),
            pltpu.VMEM((8, 4, 8, 128), jnp.float32),
            pltpu.SemaphoreType.DMA,
            pltpu.SemaphoreType.DMA,
            pltpu.SemaphoreType.DMA,
        ],
        compiler_params=pltpu.CompilerParams(use_tc_tiling_on_sc=False,
                                             needs_layout_passes=False),
    )
    def gather_kernel(p2, xs, out5, xv, g0, g1, ob, sg0, sg1, so):
        w = lax.axis_index("s") * info.num_cores + lax.axis_index("c")
        b0 = w * _BPW
        gbuf = (g0, g1)
        sg = (sg0, sg1)
        iota = lax.iota(jnp.int32, 16)

        pltpu.sync_copy(xs.at[:, pl.ds(b0, _BPW)], xv)

        def start_gather(s, h):
            pltpu.async_copy(p2.at[xv.at[s, pl.ds(256 * h, 256)]],
                             gbuf[h], sg[h])

        def wait_gather(h):
            pltpu.make_async_copy(p2.at[pl.ds(0, 256)], gbuf[h],
                                  sg[h]).wait()

        def start_store(s):
            pltpu.async_copy(ob, out5.at[s, :, pl.ds(4 * w, 4)], so)

        def wait_store():
            pltpu.make_async_copy(ob, out5.at[0, :, pl.ds(0, 4)], so).wait()

        def transform(h, g, o):
            def dt_body(dt, carry):
                for r in range(8):
                    col = jnp.full((16,), 8 * dt + r, jnp.int32)
                    for btl2 in (0, 1):
                        for c in range(8):
                            rows = iota + (128 * btl2 + 16 * c)
                            vals = plsc.load_gather(g, [rows, col])
                            o[dt, 2 * h + btl2, r, pl.ds(16 * c, 16)] = vals
                return carry

            lax.fori_loop(0, 8, dt_body, 0)

        start_gather(0, 0)

        def step(s, carry):
            start_gather(s, 1)
            wait_gather(0)

            @pl.when(s >= 1)
            def _():
                wait_store()

            transform(0, gbuf[0], ob)

            @pl.when(s + 1 < _S)
            def _():
                start_gather(s + 1, 0)

            wait_gather(1)
            transform(1, gbuf[1], ob)
            start_store(s)
            return carry

        lax.fori_loop(0, _S, step, 0)
        wait_store()

    return gather_kernel


def kernel(x, table):
    table_t = jnp.transpose(table)
    p2 = _make_pack()(table_t).reshape(_V, _D)
    xs = jnp.transpose(x).astype(jnp.int32)
    out5 = _make_gather()(p2, xs)
    t2 = jnp.transpose(out5, (2, 4, 0, 1, 3))
    return t2.reshape(_BS, _S, _D)


# speedup vs baseline: 1.1381x; 1.1381x over previous
import functools

import jax
import jax.numpy as jnp
from jax import lax
from jax.experimental import pallas as pl
from jax.experimental.pallas import tpu as pltpu
from jax.experimental.pallas import tpu_sc as plsc

_V = 1000000
_D = 64
_W = 512
_NBLK = 63
_TAIL = _V - 31 * _NBLK * _W


def _make_pack():
    info = plsc.get_sparse_core_info()
    mesh = plsc.VectorSubcoreMesh(core_axis_name="c", subcore_axis_name="s")

    @functools.partial(
        pl.kernel,
        mesh=mesh,
        out_type=jax.ShapeDtypeStruct((_V // 2, 128), jnp.float32),
        scratch_types=[
            pltpu.VMEM((_D, _W), jnp.float32),
            pltpu.VMEM((_D, _W), jnp.float32),
            pltpu.VMEM((_W // 2, 128), jnp.float32),
            pltpu.VMEM((_D, _TAIL), jnp.float32),
            pltpu.VMEM((_TAIL // 2, 128), jnp.float32),
            pltpu.SemaphoreType.DMA,
            pltpu.SemaphoreType.DMA,
            pltpu.SemaphoreType.DMA,
        ],
        compiler_params=pltpu.CompilerParams(use_tc_tiling_on_sc=True,
                                             needs_layout_passes=False),
    )
    def pack_kernel(t_t, p1, in0, in1, outb, tin, tout, si0, si1, so):
        w = lax.axis_index("s") * info.num_cores + lax.axis_index("c")
        iota = lax.iota(jnp.int32, 16)
        rowvecs = [iota + 16 * c for c in range(4)]

        def transform(src, dst, np_rows):
            def body(p, carry):
                for h in (0, 1):
                    col = jnp.full((16,), 2 * p + h, jnp.int32)
                    for c in range(4):
                        vals = plsc.load_gather(src, [rowvecs[c], col])
                        dst[p, pl.ds(64 * h + 16 * c, 16)] = vals
                return carry

            lax.fori_loop(0, np_rows, body, 0)

        @pl.when(w < 31)
        def _main():
            base = w * _NBLK
            ins = (in0, in1)
            sis = (si0, si1)

            def start_in(t, b):
                pltpu.async_copy(
                    t_t.at[:, pl.ds((base + t) * _W, _W)], ins[b], sis[b])

            def wait_in(b):
                pltpu.make_async_copy(
                    t_t.at[:, pl.ds(0, _W)], ins[b], sis[b]).wait()

            def wait_out():
                pltpu.make_async_copy(
                    outb, p1.at[pl.ds(0, _W // 2)], so).wait()

            start_in(0, 0)
            start_in(1, 1)

            def pair(k, carry):
                for bpar in (0, 1):
                    t = 2 * k + bpar
                    wait_in(bpar)

                    @pl.when(t > 0)
                    def _():
                        wait_out()

                    transform(ins[bpar], outb, _W // 2)

                    @pl.when(t + 2 < _NBLK)
                    def _():
                        start_in(t + 2, bpar)

                    pltpu.async_copy(
                        outb, p1.at[pl.ds((base + t) * (_W // 2), _W // 2)],
                        so)
                return carry

            lax.fori_loop(0, (_NBLK - 1) // 2, pair, 0)
            t = _NBLK - 1
            wait_in(t % 2)
            wait_out()
            transform(ins[t % 2], outb, _W // 2)
            pltpu.async_copy(
                outb, p1.at[pl.ds((base + t) * (_W // 2), _W // 2)], so)
            wait_out()

        @pl.when(w == 31)
        def _tail():
            pltpu.sync_copy(t_t.at[:, pl.ds(31 * _NBLK * _W, _TAIL)], tin)
            transform(tin, tout, _TAIL // 2)
            pltpu.sync_copy(tout, p1.at[pl.ds(31 * _NBLK * _W // 2,
                                              _TAIL // 2)])

    return pack_kernel


_BS = 16384
_S = 50
_BPW = 512


def _make_gather():
    info = plsc.get_sparse_core_info()
    mesh = plsc.VectorSubcoreMesh(core_axis_name="c", subcore_axis_name="s")

    @functools.partial(
        pl.kernel,
        mesh=mesh,
        out_type=jax.ShapeDtypeStruct((_S, 8, 128, 8, 128), jnp.float32),
        scratch_types=[
            pltpu.VMEM((2, _BPW), jnp.int32),
            pltpu.VMEM((_BPW // 2, _D), jnp.float32),
            pltpu.VMEM((_BPW // 2, _D), jnp.float32),
            pltpu.VMEM((_BPW // 2, 72), jnp.float32),
            pltpu.VMEM((_BPW // 2, 72), jnp.float32),
            pltpu.VMEM((8, 4, 8, 128), jnp.float32),
            pltpu.SemaphoreType.DMA,
            pltpu.SemaphoreType.DMA,
            pltpu.SemaphoreType.DMA,
            pltpu.SemaphoreType.DMA,
        ],
        compiler_params=pltpu.CompilerParams(use_tc_tiling_on_sc=False,
                                             needs_layout_passes=False),
    )
    def gather_kernel(p2, xs, out5, xv2, g0, g1, p0, p1, ob,
                      sg0, sg1, so, sx):
        w = lax.axis_index("s") * info.num_cores + lax.axis_index("c")
        b0 = w * _BPW
        gbuf = (g0, g1)
        pbuf = (p0, p1)
        sg = (sg0, sg1)
        iota = lax.iota(jnp.int32, 16)

        def start_xload(s, sp):
            pltpu.async_copy(xs.at[s, pl.ds(b0, _BPW)], xv2.at[sp], sx)

        def wait_xload():
            pltpu.make_async_copy(xs.at[0, pl.ds(0, _BPW)], xv2.at[0],
                                  sx).wait()

        def start_gather(s_row, h):
            pltpu.async_copy(p2.at[xv2.at[s_row, pl.ds(256 * h, 256)]],
                             gbuf[h], sg[h])

        def wait_gather(h):
            pltpu.make_async_copy(p2.at[pl.ds(0, 256)], gbuf[h],
                                  sg[h]).wait()

        def fill_pad(g, pb):
            def j_body(i, carry):
                for u in range(4):
                    for c in range(4):
                        sl = pl.ds(16 * c, 16)
                        pb[4 * i + u, sl] = g[4 * i + u, sl]
                return carry

            lax.fori_loop(0, _BPW // 8, j_body, 0)

        def start_store(s):
            pltpu.async_copy(ob, out5.at[s, :, pl.ds(4 * w, 4)], so)

        def wait_store():
            pltpu.make_async_copy(ob, out5.at[0, :, pl.ds(0, 4)], so).wait()

        def transform(h, g, o):
            def dt_body(dt, carry):
                for r in range(8):
                    col = jnp.full((16,), 8 * dt + r, jnp.int32)
                    for btl2 in (0, 1):
                        for c in range(8):
                            rows = iota + (128 * btl2 + 16 * c)
                            vals = plsc.load_gather(g, [rows, col])
                            o[dt, 2 * h + btl2, r, pl.ds(16 * c, 16)] = vals
                return carry

            lax.fori_loop(0, 8, dt_body, 0)

        pltpu.sync_copy(xs.at[0, pl.ds(b0, _BPW)], xv2.at[0])
        start_gather(0, 0)
        start_xload(1, 1)

        def pair(k, carry):
            for sp in (0, 1):
                s = 2 * k + sp
                start_gather(sp, 1)
                wait_gather(0)
                fill_pad(gbuf[0], pbuf[0])

                @pl.when(s + 1 < _S)
                def _():
                    wait_xload()
                    start_gather(1 - sp, 0)

                @pl.when(s + 2 < _S)
                def _():
                    start_xload(s + 2, sp)

                @pl.when(s >= 1)
                def _():
                    wait_store()

                transform(0, pbuf[0], ob)
                wait_gather(1)
                fill_pad(gbuf[1], pbuf[1])
                transform(1, pbuf[1], ob)
                start_store(s)
            return carry

        lax.fori_loop(0, _S // 2, pair, 0)
        wait_store()

    return gather_kernel


def kernel(x, table):
    table_t = jnp.transpose(table)
    p2 = _make_pack()(table_t).reshape(_V, _D)
    xs = jnp.transpose(x).astype(jnp.int32)
    out5 = _make_gather()(p2, xs)
    t2 = jnp.transpose(out5, (2, 4, 0, 1, 3))
    return t2.reshape(_BS, _S, _D)
